# baseline (speedup 1.0000x reference)
import functools

import jax
import jax.numpy as jnp
from jax import lax
from jax.experimental import pallas as pl
from jax.experimental.pallas import tpu as pltpu
from jax.experimental.pallas import tpu_sc as plsc

NC = 2
NS = 16
NW = NC * NS

B = 1024
L = 200
D = 64
V = 1_000_000

WB = 256
NB = V // WB
VMAIN = NB * WB
PKROWS = VMAIN // 2
NB_W = NB // NW
NB_X = NB - NW * NB_W
CHUNK = 128
NCH = B // CHUNK

_params = pltpu.CompilerParams(use_tc_tiling_on_sc=True, needs_layout_passes=False)


def _mesh():
    return plsc.VectorSubcoreMesh(
        core_axis_name="c", subcore_axis_name="s", num_cores=NC, num_subcores=NS
    )


def _wid():
    return lax.axis_index("s") * NC + lax.axis_index("c")


def _transpose_block(in_v, out_v):
    iota = lax.iota(jnp.int32, 16)
    d_vecs = [iota + 16 * m for m in range(4)]

    def v2_body(v2, _):
        e = 2 * v2
        vls = [jnp.full((16,), e, jnp.int32), jnp.full((16,), e + 1, jnp.int32)]
        vals = [plsc.load_gather(in_v, [d_vecs[k % 4], vls[k // 4]])
                for k in range(8)]
        for k in range(8):
            out_v[v2, pl.ds(16 * k, 16)] = vals[k]
        return ()

    lax.fori_loop(0, WB // 2, v2_body, (), unroll=2)


def _pack_body(tok_t, pk, in_a, in_b, out_a, out_b, si_a, si_b, so_a, so_b):
    wid = _wid()
    base = wid * NB_W

    def start_in(i, buf, sem):
        pltpu.async_copy(tok_t.at[:, pl.ds(i * WB, WB)], buf.at[:, pl.ds(0, WB)], sem)

    def start_out(i, buf, sem):
        pltpu.async_copy(buf, pk.at[pl.ds(i * (WB // 2), WB // 2), :], sem)

    def wait_in(buf, sem):
        pltpu.make_async_copy(tok_t.at[:, pl.ds(0, WB)], buf.at[:, pl.ds(0, WB)], sem).wait()

    def wait_out(buf, sem):
        pltpu.make_async_copy(buf, pk.at[pl.ds(0, WB // 2), :], sem).wait()

    start_in(base, in_a, si_a)

    def u_body(u, _):
        t_a = base + 2 * u
        start_in(t_a + 1, in_b, si_b)
        with jax.named_scope("pk_wait_in_a"):
            wait_in(in_a, si_a)

        @pl.when(u > 0)
        def _():
            with jax.named_scope("pk_wait_out_a"):
                wait_out(out_a, so_a)

        with jax.named_scope("pk_transpose_a"):
            _transpose_block(in_a, out_a)
        start_out(t_a, out_a, so_a)

        @pl.when(u < NB_W // 2 - 1)
        def _():
            start_in(t_a + 2, in_a, si_a)

        wait_in(in_b, si_b)

        @pl.when(u > 0)
        def _():
            wait_out(out_b, so_b)

        _transpose_block(in_b, out_b)
        start_out(t_a + 1, out_b, so_b)
        return ()

    lax.fori_loop(0, NB_W // 2, u_body, ())

    @pl.when(wid < NB_X)
    def _extra():
        i = NW * NB_W + wid
        wait_out(out_a, so_a)
        pltpu.sync_copy(tok_t.at[:, pl.ds(i * WB, WB)], in_a.at[:, pl.ds(0, WB)])
        _transpose_block(in_a, out_a)
        start_out(i, out_a, so_a)

    @pl.when(wid >= NB_X)
    def _():
        wait_out(out_a, so_a)

    @pl.when(wid < NB_X)
    def _():
        wait_out(out_a, so_a)

    wait_out(out_b, so_b)


def _gather_body(pk, x_t, pos_hbm, aux_hbm, out3,
                 pos_v, g_a, g_b, ostage, xi_l, idx2_l,
                 sg_a, sg_b):
    wid = _wid()
    iota = lax.iota(jnp.int32, 16)

    pltpu.sync_copy(pos_hbm, pos_v)
    pltpu.sync_copy(aux_hbm, g_a.at[pl.ds(CHUNK, 32), :])
    pltpu.sync_copy(aux_hbm, g_b.at[pl.ds(CHUNK, 32), :])
    nl = jnp.where(wid < L - NW * (L // NW), L // NW + 1, L // NW)

    def start_g(c, buf, sem):
        pltpu.async_copy(
            pk.at[idx2_l.at[pl.ds(CHUNK * c, CHUNK)]],
            buf.at[pl.ds(0, CHUNK), :], sem)

    def wait_g(buf, sem):
        pltpu.make_async_copy(
            pk.at[idx2_l.at[pl.ds(0, CHUNK)]],
            buf.at[pl.ds(0, CHUNK), :], sem).wait()

    def l_body(j, _):
        l = wid + NW * j
        with jax.named_scope("g_xi"):
            pltpu.sync_copy(x_t.at[l], xi_l.at[pl.ds(0, B)])

        def prep(k, _):
            sl = pl.ds(16 * k, 16)
            xi = xi_l[sl]
            idx2 = lax.shift_right_logical(xi, 1)
            idx2_l[sl] = jnp.minimum(idx2, PKROWS - 1)
            return ()

        with jax.named_scope("g_prep"):
            lax.fori_loop(0, B // 16, prep, ())
        start_g(0, g_a, sg_a)

        pos_m = [pos_v[pl.ds(l * D + 16 * m, 16)] for m in range(D // 16)]
        d_vecs = [iota + 16 * m for m in range(D // 16)]

        def compute(buf, c):
            cb = CHUNK * c

            def t_body(r, _):
                xi = xi_l[pl.ds(cb + r, 16)][0]
                idx2 = lax.shift_right_logical(xi, 1)
                off = lax.shift_left(jnp.bitwise_and(xi, 1), 6)
                row = jnp.where(idx2 >= PKROWS, idx2 - (PKROWS - CHUNK), r)
                colv = jnp.full((16,), cb + r, jnp.int32)
                vals = [buf[row, pl.ds(off + 16 * m, 16)]
                        for m in range(D // 16)]
                for m in range(D // 16):
                    plsc.store_scatter(ostage, [d_vecs[m], colv],
                                       vals[m] + pos_m[m])
                return ()

            lax.fori_loop(0, CHUNK, t_body, (), unroll=4)

        def c_body(p, _):
            start_g(2 * p + 1, g_b, sg_b)
            with jax.named_scope("g_wait_a"):
                wait_g(g_a, sg_a)
            with jax.named_scope("g_compute_a"):
                compute(g_a, 2 * p)

            @pl.when(p < NCH // 2 - 1)
            def _():
                start_g(2 * p + 2, g_a, sg_a)

            wait_g(g_b, sg_b)
            compute(g_b, 2 * p + 1)
            return ()

        lax.fori_loop(0, NCH // 2, c_body, ())
        with jax.named_scope("g_out"):
            pltpu.sync_copy(ostage.at[:, pl.ds(0, B)], out3.at[l])
        return ()

    lax.fori_loop(0, nl, l_body, ())


@jax.jit
def _run(tok_t, x_t, pos_flat, aux_pairs):
    pack = pl.kernel(
        _pack_body,
        out_type=jax.ShapeDtypeStruct((PKROWS, 128), jnp.float32),
        mesh=_mesh(),
        scratch_types=[
            pltpu.VMEM((D, WB + 5), jnp.float32),
            pltpu.VMEM((D, WB + 5), jnp.float32),
            pltpu.VMEM((WB // 2, 128), jnp.float32),
            pltpu.VMEM((WB // 2, 128), jnp.float32),
            pltpu.SemaphoreType.DMA,
            pltpu.SemaphoreType.DMA,
            pltpu.SemaphoreType.DMA,
            pltpu.SemaphoreType.DMA,
        ],
        compiler_params=_params,
    )
    pk = pack(tok_t)

    gather = pl.kernel(
        _gather_body,
        out_type=jax.ShapeDtypeStruct((L, D, B), jnp.float32),
        mesh=_mesh(),
        scratch_types=[
            pltpu.VMEM((L * D,), jnp.float32),
            pltpu.VMEM((CHUNK + 32, 128), jnp.float32),
            pltpu.VMEM((CHUNK + 32, 128), jnp.float32),
            pltpu.VMEM((D, B + 1), jnp.float32),
            pltpu.VMEM((B + 16,), jnp.int32),
            pltpu.VMEM((B,), jnp.int32),
            pltpu.SemaphoreType.DMA,
            pltpu.SemaphoreType.DMA,
        ],
        compiler_params=_params,
    )
    out3 = gather(pk, x_t, pos_flat, aux_pairs)
    return out3.transpose(2, 0, 1)


def kernel(x, token_table, pos_table):
    tok_t = token_table.T
    x_t = x.T.astype(jnp.int32)
    pos_flat = pos_table.reshape(-1)
    aux_pairs = lax.slice(token_table, (VMAIN, 0), (V, D)).reshape(32, 128)
    return _run(tok_t, x_t, pos_flat, aux_pairs)


# speedup vs baseline: 1.8494x; 1.8494x over previous
import functools

import jax
import jax.numpy as jnp
from jax import lax
from jax.experimental import pallas as pl
from jax.experimental.pallas import tpu as pltpu
from jax.experimental.pallas import tpu_sc as plsc

NC = 2
NS = 16
NW = NC * NS
LANES = 16

B = 1024
L = 200
D = 64
ROWS_PER_WORKER = B // NW


def _embed_body(x_hbm, tok_hbm, pos_hbm, out_hbm,
                idx_a, idx_b, rows_a, rows_b, pos_v, sg_a, sg_b, so_a, so_b):
    wid = lax.axis_index("s") * NC + lax.axis_index("c")
    pltpu.sync_copy(pos_hbm, pos_v)
    base0 = wid * ROWS_PER_WORKER * L

    def start(i, idx_v, rows_v, sem):
        b0 = base0 + i * L
        pltpu.sync_copy(x_hbm.at[pl.ds(b0, L)], idx_v)
        pltpu.async_copy(tok_hbm.at[idx_v], rows_v, sem)

    def wait_g(rows_v, sem):
        pltpu.make_async_copy(tok_hbm.at[idx_a], rows_v, sem).wait()

    def wait_o(rows_v, sem):
        pltpu.make_async_copy(rows_v, out_hbm.at[pl.ds(0, L)], sem).wait()

    def add_store(i, rows_v, sem):
        def row_body(r, _):
            for j in range(D // LANES):
                sl = pl.ds(j * LANES, LANES)
                rows_v[r, sl] = rows_v[r, sl] + pos_v[r, sl]
            return ()

        lax.fori_loop(0, L, row_body, (), unroll=2)
        pltpu.async_copy(rows_v, out_hbm.at[pl.ds(base0 + i * L, L)], sem)

    start(0, idx_a, rows_a, sg_a)

    def u_body(u, _):
        i_a = 2 * u
        start(i_a + 1, idx_b, rows_b, sg_b)
        wait_g(rows_a, sg_a)

        @pl.when(u > 0)
        def _():
            wait_o(rows_a, so_a)

        add_store(i_a, rows_a, so_a)

        @pl.when(u < ROWS_PER_WORKER // 2 - 1)
        def _():
            start(i_a + 2, idx_a, rows_a, sg_a)

        wait_g(rows_b, sg_b)

        @pl.when(u > 0)
        def _():
            wait_o(rows_b, so_b)

        add_store(i_a + 1, rows_b, so_b)
        return ()

    lax.fori_loop(0, ROWS_PER_WORKER // 2, u_body, ())
    wait_o(rows_a, so_a)
    wait_o(rows_b, so_b)


@jax.jit
def _embed(x_flat, token_table, pos_table):
    mesh = plsc.VectorSubcoreMesh(
        core_axis_name="c", subcore_axis_name="s", num_cores=NC, num_subcores=NS
    )
    run = pl.kernel(
        _embed_body,
        out_type=jax.ShapeDtypeStruct((B * L, D), jnp.float32),
        mesh=mesh,
        scratch_types=[
            pltpu.VMEM((L,), jnp.int32),
            pltpu.VMEM((L,), jnp.int32),
            pltpu.VMEM((L, D), jnp.float32),
            pltpu.VMEM((L, D), jnp.float32),
            pltpu.VMEM((L, D), jnp.float32),
            pltpu.SemaphoreType.DMA,
            pltpu.SemaphoreType.DMA,
            pltpu.SemaphoreType.DMA,
            pltpu.SemaphoreType.DMA,
        ],
        compiler_params=pltpu.CompilerParams(use_tc_tiling_on_sc=False),
    )
    out = run(x_flat, token_table, pos_table)
    return out.reshape(B, L, D)


def kernel(x, token_table, pos_table):
    x_flat = x.reshape(-1).astype(jnp.int32)
    return _embed(x_flat, token_table, pos_table)


# speedup vs baseline: 1.8830x; 1.0182x over previous
import functools

import jax
import jax.numpy as jnp
from jax import lax
from jax.experimental import pallas as pl
from jax.experimental.pallas import tpu as pltpu
from jax.experimental.pallas import tpu_sc as plsc

NC = 2
NS = 16
NW = NC * NS
LANES = 16

B = 1024
L = 200
D = 64
ROWS_PER_WORKER = B // NW


def _embed_body(x_hbm, tok_hbm, pos_hbm, out_hbm,
                idx_all, rows_a, rows_b, pos_v, sg_a, sg_b, so_a, so_b):
    wid = lax.axis_index("s") * NC + lax.axis_index("c")
    pltpu.sync_copy(pos_hbm, pos_v)
    base0 = wid * ROWS_PER_WORKER * L
    pltpu.sync_copy(x_hbm.at[pl.ds(base0, ROWS_PER_WORKER * L)], idx_all)

    def start(i, rows_v, sem):
        pltpu.async_copy(
            tok_hbm.at[idx_all.at[pl.ds(i * L, L)]], rows_v, sem)

    def wait_g(rows_v, sem):
        pltpu.make_async_copy(
            tok_hbm.at[idx_all.at[pl.ds(0, L)]], rows_v, sem).wait()

    def wait_o(rows_v, sem):
        pltpu.make_async_copy(rows_v, out_hbm.at[pl.ds(0, L)], sem).wait()

    def add_store(i, rows_v, sem):
        def row_body(r, _):
            for j in range(D // LANES):
                sl = pl.ds(j * LANES, LANES)
                rows_v[r, sl] = rows_v[r, sl] + pos_v[r, sl]
            return ()

        lax.fori_loop(0, L, row_body, (), unroll=2)
        pltpu.async_copy(rows_v, out_hbm.at[pl.ds(base0 + i * L, L)], sem)

    start(0, rows_a, sg_a)

    def u_body(u, _):
        i_a = 2 * u
        start(i_a + 1, rows_b, sg_b)
        wait_g(rows_a, sg_a)

        @pl.when(u > 0)
        def _():
            wait_o(rows_a, so_a)

        add_store(i_a, rows_a, so_a)

        @pl.when(u < ROWS_PER_WORKER // 2 - 1)
        def _():
            start(i_a + 2, rows_a, sg_a)

        wait_g(rows_b, sg_b)

        @pl.when(u > 0)
        def _():
            wait_o(rows_b, so_b)

        add_store(i_a + 1, rows_b, so_b)
        return ()

    lax.fori_loop(0, ROWS_PER_WORKER // 2, u_body, ())
    wait_o(rows_a, so_a)
    wait_o(rows_b, so_b)


@jax.jit
def _embed(x_flat, token_table, pos_table):
    mesh = plsc.VectorSubcoreMesh(
        core_axis_name="c", subcore_axis_name="s", num_cores=NC, num_subcores=NS
    )
    run = pl.kernel(
        _embed_body,
        out_type=jax.ShapeDtypeStruct((B * L, D), jnp.float32),
        mesh=mesh,
        scratch_types=[
            pltpu.VMEM((ROWS_PER_WORKER * L,), jnp.int32),
            pltpu.VMEM((L, D), jnp.float32),
            pltpu.VMEM((L, D), jnp.float32),
            pltpu.VMEM((L, D), jnp.float32),
            pltpu.SemaphoreType.DMA,
            pltpu.SemaphoreType.DMA,
            pltpu.SemaphoreType.DMA,
            pltpu.SemaphoreType.DMA,
        ],
        compiler_params=pltpu.CompilerParams(use_tc_tiling_on_sc=False),
    )
    out = run(x_flat, token_table, pos_table)
    return out.reshape(B, L, D)


def kernel(x, token_table, pos_table):
    x_flat = x.reshape(-1).astype(jnp.int32)
    return _embed(x_flat, token_table, pos_table)
